# baseline (speedup 1.0000x reference)
import functools

import jax
import jax.numpy as jnp
import numpy as np
from jax import lax
from jax.experimental import pallas as pl
from jax.experimental.pallas import tpu as pltpu
from jax.experimental.pallas import tpu_sc as plsc

MAX_LEN = 200
EMB_DIM = 64
BATCH = 4096
N_VOCAB = 1000000
N_UNITS = N_VOCAB // 2

NUM_CORES = 2
NUM_SUBCORES = 16
NUM_WORKERS = NUM_CORES * NUM_SUBCORES

VCHUNK = 128
N_CHUNKS = (N_VOCAB + VCHUNK - 1) // VCHUNK
CHUNKS_PER_W = N_CHUNKS // NUM_WORKERS
CHUNK_REM = N_CHUNKS - CHUNKS_PER_W * NUM_WORKERS

BLOCK = 128
NBLK = BATCH // BLOCK


def _pe_const():
    pos = np.expand_dims(np.arange(MAX_LEN), 1)
    pe = pos / np.power(1000, 2 * np.expand_dims(np.arange(EMB_DIM) // 2, 0) / EMB_DIM)
    pe[:, 0::2] = np.sin(pe[:, 0::2])
    pe[:, 1::2] = np.cos(pe[:, 1::2])
    return pe.astype(np.float32)


_PE = _pe_const()


@functools.partial(
    pl.kernel,
    out_type=jax.ShapeDtypeStruct((N_UNITS, 2 * EMB_DIM), jnp.float32),
    mesh=plsc.VectorSubcoreMesh(core_axis_name="c", subcore_axis_name="s"),
    scratch_types=[
        [pltpu.VMEM((EMB_DIM, VCHUNK), jnp.float32) for _ in range(2)],
        [pltpu.VMEM((VCHUNK // 2, 2 * EMB_DIM), jnp.float32) for _ in range(2)],
        pltpu.SemaphoreType.DMA,
        pltpu.SemaphoreType.DMA,
    ],
    compiler_params=pltpu.CompilerParams(needs_layout_passes=False),
)
def _detile(tabt_hbm, dup_hbm, srcs, rms, isem, osem):
    wid = lax.axis_index("s") * NUM_CORES + lax.axis_index("c")
    base = wid * CHUNKS_PER_W + jnp.minimum(wid, CHUNK_REM)
    cnt = CHUNKS_PER_W + jnp.where(wid < CHUNK_REM, 1, 0)

    iota16 = lax.iota(jnp.int32, 16)
    rows_j = [iota16 + 16 * j for j in range(EMB_DIM // 16)]

    def chunk_in_start(c, src):
        pltpu.async_copy(tabt_hbm.at[:, pl.ds(c * VCHUNK, VCHUNK)], src, isem)

    def chunk_in_wait(src):
        pltpu.make_async_copy(tabt_hbm.at[:, pl.ds(0, VCHUNK)], src, isem).wait()

    def shuffle(src, rm):
        @plsc.parallel_loop(0, 16, unroll=4)
        def _(k):
            dk = (iota16 + k) % 16
            for g in range(VCHUNK // 16):
                v_vec = dk + 16 * g
                urow = v_vec >> 1
                ucol0 = (v_vec & 1) * EMB_DIM + iota16 * 0
                for j in range(EMB_DIM // 16):
                    val = plsc.load_gather(src, [rows_j[j], v_vec])
                    plsc.store_scatter(rm, [urow, ucol0 + 16 * j + iota16], val)

    def out_start(c, rm):
        @pl.when(c < N_CHUNKS - 1)
        def _():
            pltpu.async_copy(
                rm, dup_hbm.at[pl.ds(c * (VCHUNK // 2), VCHUNK // 2), :], osem
            )

        @pl.when(c == N_CHUNKS - 1)
        def _():
            pltpu.async_copy(
                rm.at[pl.ds(0, VCHUNK // 4), :],
                dup_hbm.at[pl.ds(c * (VCHUNK // 2), VCHUNK // 4), :],
                osem,
            )

    def out_wait(c, rm):
        @pl.when(c < N_CHUNKS - 1)
        def _():
            pltpu.make_async_copy(
                rm, dup_hbm.at[pl.ds(0, VCHUNK // 2), :], osem
            ).wait()

        @pl.when(c == N_CHUNKS - 1)
        def _():
            pltpu.make_async_copy(
                rm.at[pl.ds(0, VCHUNK // 4), :],
                dup_hbm.at[pl.ds(0, VCHUNK // 4), :],
                osem,
            ).wait()

    chunk_in_start(base, srcs[0])

    def body(i, carry):
        for p in range(2):
            k = 2 * i + p
            c = base + k

            @pl.when(k < cnt)
            def _():
                @pl.when(k + 1 < cnt)
                def _():
                    chunk_in_start(c + 1, srcs[(p + 1) % 2])

                chunk_in_wait(srcs[p])

                @pl.when(k >= 2)
                def _():
                    out_wait(c - 2, rms[p])

                shuffle(srcs[p], rms[p])
                out_start(c, rms[p])

        return carry

    lax.fori_loop(0, (CHUNKS_PER_W + 2) // 2, body, 0)

    @pl.when(cnt % 2 == 0)
    def _():
        out_wait(base + cnt - 2, rms[0])
        out_wait(base + cnt - 1, rms[1])

    @pl.when(cnt % 2 == 1)
    def _():
        out_wait(base + cnt - 2, rms[1])
        out_wait(base + cnt - 1, rms[0])


@functools.partial(
    pl.kernel,
    out_type=jax.ShapeDtypeStruct((MAX_LEN, EMB_DIM, BATCH), jnp.float32),
    mesh=plsc.VectorSubcoreMesh(core_axis_name="c", subcore_axis_name="s"),
    scratch_types=[
        pltpu.VMEM((BATCH,), jnp.int32),
        pltpu.VMEM((BATCH,), jnp.int32),
        pltpu.VMEM((MAX_LEN, EMB_DIM), jnp.float32),
        [pltpu.VMEM((BLOCK, 2 * EMB_DIM), jnp.float32) for _ in range(2)],
        [pltpu.VMEM((EMB_DIM, BLOCK), jnp.float32) for _ in range(2)],
        pltpu.SemaphoreType.DMA,
        pltpu.SemaphoreType.DMA,
    ],
    compiler_params=pltpu.CompilerParams(needs_layout_passes=False),
)
def _gather(
    xf_hbm, tab_hbm, pe_hbm, out_hbm, idx_v, unit_v, pe_v, bufs, bufts, gsem, osem
):
    wid = lax.axis_index("s") * NUM_CORES + lax.axis_index("c")
    t_lo = wid * MAX_LEN // NUM_WORKERS
    t_hi = (wid + 1) * MAX_LEN // NUM_WORKERS
    pltpu.sync_copy(pe_hbm, pe_v)

    iota16 = lax.iota(jnp.int32, 16)

    def gather_start(bi, buf):
        pltpu.async_copy(tab_hbm.at[unit_v.at[pl.ds(bi * BLOCK, BLOCK)]], buf, gsem)

    def gather_wait(buf):
        pltpu.make_async_copy(
            tab_hbm.at[unit_v.at[pl.ds(0, BLOCK)]], buf, gsem
        ).wait()

    def process(bi, buf, buft, t):
        par64 = []
        rows_g = []
        for g in range(BLOCK // 16):
            vg = idx_v[pl.ds(bi * BLOCK + 16 * g, 16)]
            par64.append((vg & 1) * EMB_DIM)
            rows_g.append(iota16 + 16 * g)

        @plsc.parallel_loop(0, 16, unroll=4)
        def _(k):
            dk = (iota16 + k) % 16
            for j in range(EMB_DIM // 16):
                dcols = dk + 16 * j
                pe_diag = plsc.load_gather(pe_v, [iota16 * 0 + t, dcols])
                for g in range(BLOCK // 16):
                    val = plsc.load_gather(buf, [rows_g[g], dcols + par64[g]])
                    plsc.store_scatter(buft, [dcols, rows_g[g]], val + pe_diag)

    def out_start(bi, buft, t):
        pltpu.async_copy(buft, out_hbm.at[t, :, pl.ds(bi * BLOCK, BLOCK)], osem)

    def out_wait(buft, t):
        pltpu.make_async_copy(
            buft, out_hbm.at[t, :, pl.ds(0, BLOCK)], osem
        ).wait()

    def t_body(t, carry):
        pltpu.sync_copy(xf_hbm.at[pl.ds(t * BATCH, BATCH)], idx_v)

        def half_body(q, c):
            for u in range(8):
                off = (q * 8 + u) * 16
                unit_v[pl.ds(off, 16)] = idx_v[pl.ds(off, 16)] >> 1
            return c

        lax.fori_loop(0, BATCH // 128, half_body, 0)
        gather_start(0, bufs[0])

        def blk_body(i, c):
            for p in range(2):
                bi = 2 * i + p

                @pl.when(bi + 1 < NBLK)
                def _():
                    gather_start(bi + 1, bufs[(p + 1) % 2])

                gather_wait(bufs[p])

                @pl.when(bi >= 2)
                def _():
                    out_wait(bufts[p], t)

                process(bi, bufs[p], bufts[p], t)
                out_start(bi, bufts[p], t)
            return c

        lax.fori_loop(0, NBLK // 2, blk_body, 0)
        out_wait(bufts[0], t)
        out_wait(bufts[1], t)
        return carry

    lax.fori_loop(t_lo, t_hi, t_body, 0)


def kernel(x, table):
    xf = jnp.transpose(x).reshape(-1).astype(jnp.int32)
    tabt = jnp.transpose(table)
    packed = _detile(tabt)
    out3 = _gather(xf, packed, jnp.asarray(_PE))
    return jnp.transpose(out3, (2, 0, 1))


# speedup vs baseline: 1.1452x; 1.1452x over previous
import functools

import jax
import jax.numpy as jnp
import numpy as np
from jax import lax
from jax.experimental import pallas as pl
from jax.experimental.pallas import tpu as pltpu
from jax.experimental.pallas import tpu_sc as plsc

MAX_LEN = 200
EMB_DIM = 64
BATCH = 4096
N_VOCAB = 1000000
N_UNITS = N_VOCAB // 2

NUM_CORES = 2
NUM_SUBCORES = 16
NUM_WORKERS = NUM_CORES * NUM_SUBCORES

VCHUNK = 128
N_CHUNKS = (N_VOCAB + VCHUNK - 1) // VCHUNK
CHUNKS_PER_W = N_CHUNKS // NUM_WORKERS
CHUNK_REM = N_CHUNKS - CHUNKS_PER_W * NUM_WORKERS

BLOCK = 128
NBLK = BATCH // BLOCK


def _pe_const():
    pos = np.expand_dims(np.arange(MAX_LEN), 1)
    pe = pos / np.power(1000, 2 * np.expand_dims(np.arange(EMB_DIM) // 2, 0) / EMB_DIM)
    pe[:, 0::2] = np.sin(pe[:, 0::2])
    pe[:, 1::2] = np.cos(pe[:, 1::2])
    return pe.astype(np.float32)


_PE = _pe_const()


@functools.partial(
    pl.kernel,
    out_type=jax.ShapeDtypeStruct((N_UNITS, 2 * EMB_DIM), jnp.float32),
    mesh=plsc.VectorSubcoreMesh(core_axis_name="c", subcore_axis_name="s"),
    scratch_types=[
        [pltpu.VMEM((EMB_DIM, VCHUNK), jnp.float32) for _ in range(2)],
        [pltpu.VMEM((VCHUNK // 2, 2 * EMB_DIM), jnp.float32) for _ in range(2)],
        pltpu.SemaphoreType.DMA,
        pltpu.SemaphoreType.DMA,
    ],
    compiler_params=pltpu.CompilerParams(needs_layout_passes=False),
)
def _detile(tabt_hbm, dup_hbm, srcs, rms, isem, osem):
    wid = lax.axis_index("s") * NUM_CORES + lax.axis_index("c")
    base = wid * CHUNKS_PER_W + jnp.minimum(wid, CHUNK_REM)
    cnt = CHUNKS_PER_W + jnp.where(wid < CHUNK_REM, 1, 0)

    iota16 = lax.iota(jnp.int32, 16)
    rows_j = [iota16 + 16 * j for j in range(EMB_DIM // 16)]

    def chunk_in_start(c, src):
        pltpu.async_copy(tabt_hbm.at[:, pl.ds(c * VCHUNK, VCHUNK)], src, isem)

    def chunk_in_wait(src):
        pltpu.make_async_copy(tabt_hbm.at[:, pl.ds(0, VCHUNK)], src, isem).wait()

    def shuffle(src, rm):
        @plsc.parallel_loop(0, 16, unroll=2)
        def _(k):
            dk = (iota16 + k) % 16
            for g in range(VCHUNK // 16):
                v_vec = dk + 16 * g
                urow = v_vec >> 1
                ucol0 = (v_vec & 1) * EMB_DIM + iota16 * 0
                for j in range(EMB_DIM // 16):
                    val = plsc.load_gather(src, [rows_j[j], v_vec])
                    plsc.store_scatter(rm, [urow, ucol0 + 16 * j + iota16], val)

    def out_start(c, rm):
        @pl.when(c < N_CHUNKS - 1)
        def _():
            pltpu.async_copy(
                rm, dup_hbm.at[pl.ds(c * (VCHUNK // 2), VCHUNK // 2), :], osem
            )

        @pl.when(c == N_CHUNKS - 1)
        def _():
            pltpu.async_copy(
                rm.at[pl.ds(0, VCHUNK // 4), :],
                dup_hbm.at[pl.ds(c * (VCHUNK // 2), VCHUNK // 4), :],
                osem,
            )

    def out_wait(c, rm):
        @pl.when(c < N_CHUNKS - 1)
        def _():
            pltpu.make_async_copy(
                rm, dup_hbm.at[pl.ds(0, VCHUNK // 2), :], osem
            ).wait()

        @pl.when(c == N_CHUNKS - 1)
        def _():
            pltpu.make_async_copy(
                rm.at[pl.ds(0, VCHUNK // 4), :],
                dup_hbm.at[pl.ds(0, VCHUNK // 4), :],
                osem,
            ).wait()

    chunk_in_start(base, srcs[0])

    def body(i, carry):
        for p in range(2):
            k = 2 * i + p
            c = base + k

            @pl.when(k < cnt)
            def _():
                @pl.when(k + 1 < cnt)
                def _():
                    chunk_in_start(c + 1, srcs[(p + 1) % 2])

                chunk_in_wait(srcs[p])

                @pl.when(k >= 2)
                def _():
                    out_wait(c - 2, rms[p])

                shuffle(srcs[p], rms[p])
                out_start(c, rms[p])

        return carry

    lax.fori_loop(0, (CHUNKS_PER_W + 2) // 2, body, 0)

    @pl.when(cnt % 2 == 0)
    def _():
        out_wait(base + cnt - 2, rms[0])
        out_wait(base + cnt - 1, rms[1])

    @pl.when(cnt % 2 == 1)
    def _():
        out_wait(base + cnt - 2, rms[1])
        out_wait(base + cnt - 1, rms[0])


@functools.partial(
    pl.kernel,
    out_type=jax.ShapeDtypeStruct((MAX_LEN, EMB_DIM, BATCH), jnp.float32),
    mesh=plsc.VectorSubcoreMesh(core_axis_name="c", subcore_axis_name="s"),
    scratch_types=[
        pltpu.VMEM((BATCH,), jnp.int32),
        pltpu.VMEM((BATCH,), jnp.int32),
        pltpu.VMEM((MAX_LEN, EMB_DIM), jnp.float32),
        [pltpu.VMEM((BLOCK, 2 * EMB_DIM), jnp.float32) for _ in range(2)],
        [pltpu.VMEM((EMB_DIM, BLOCK), jnp.float32) for _ in range(2)],
        pltpu.SemaphoreType.DMA,
        pltpu.SemaphoreType.DMA,
    ],
    compiler_params=pltpu.CompilerParams(needs_layout_passes=False),
)
def _gather(
    xf_hbm, tab_hbm, pe_hbm, out_hbm, idx_v, unit_v, pe_v, bufs, bufts, gsem, osem
):
    wid = lax.axis_index("s") * NUM_CORES + lax.axis_index("c")
    t_lo = wid * MAX_LEN // NUM_WORKERS
    t_hi = (wid + 1) * MAX_LEN // NUM_WORKERS
    pltpu.sync_copy(pe_hbm, pe_v)

    iota16 = lax.iota(jnp.int32, 16)

    def gather_start(bi, buf):
        pltpu.async_copy(tab_hbm.at[unit_v.at[pl.ds(bi * BLOCK, BLOCK)]], buf, gsem)

    def gather_wait(buf):
        pltpu.make_async_copy(
            tab_hbm.at[unit_v.at[pl.ds(0, BLOCK)]], buf, gsem
        ).wait()

    def process(bi, buf, buft, t):
        par64 = []
        rows_g = []
        for g in range(BLOCK // 16):
            vg = idx_v[pl.ds(bi * BLOCK + 16 * g, 16)]
            par64.append((vg & 1) * EMB_DIM)
            rows_g.append(iota16 + 16 * g)

        @plsc.parallel_loop(0, 16, unroll=2)
        def _(k):
            dk = (iota16 + k) % 16
            for j in range(EMB_DIM // 16):
                dcols = dk + 16 * j
                pe_diag = plsc.load_gather(pe_v, [iota16 * 0 + t, dcols])
                for g in range(BLOCK // 16):
                    val = plsc.load_gather(buf, [rows_g[g], dcols + par64[g]])
                    plsc.store_scatter(buft, [dcols, rows_g[g]], val + pe_diag)

    def out_start(bi, buft, t):
        pltpu.async_copy(buft, out_hbm.at[t, :, pl.ds(bi * BLOCK, BLOCK)], osem)

    def out_wait(buft, t):
        pltpu.make_async_copy(
            buft, out_hbm.at[t, :, pl.ds(0, BLOCK)], osem
        ).wait()

    def t_body(t, carry):
        pltpu.sync_copy(xf_hbm.at[pl.ds(t * BATCH, BATCH)], idx_v)

        def half_body(q, c):
            for u in range(8):
                off = (q * 8 + u) * 16
                unit_v[pl.ds(off, 16)] = idx_v[pl.ds(off, 16)] >> 1
            return c

        lax.fori_loop(0, BATCH // 128, half_body, 0)
        gather_start(0, bufs[0])

        def blk_body(i, c):
            for p in range(2):
                bi = 2 * i + p

                @pl.when(bi + 1 < NBLK)
                def _():
                    gather_start(bi + 1, bufs[(p + 1) % 2])

                gather_wait(bufs[p])

                @pl.when(bi >= 2)
                def _():
                    out_wait(bufts[p], t)

                process(bi, bufs[p], bufts[p], t)
                out_start(bi, bufts[p], t)
            return c

        lax.fori_loop(0, NBLK // 2, blk_body, 0)
        out_wait(bufts[0], t)
        out_wait(bufts[1], t)
        return carry

    lax.fori_loop(t_lo, t_hi, t_body, 0)


def kernel(x, table):
    xf = jnp.transpose(x).reshape(-1).astype(jnp.int32)
    tabt = jnp.transpose(table)
    packed = _detile(tabt)
    out3 = _gather(xf, packed, jnp.asarray(_PE))
    return jnp.transpose(out3, (2, 0, 1))
